# baseline (speedup 1.0000x reference)
import functools

import numpy as np
import jax
import jax.numpy as jnp
from jax import lax
from jax.experimental import pallas as pl
from jax.experimental.pallas import tpu as pltpu
from jax.experimental.pallas import tpu_sc as plsc

M = 16384
U = 1024
NC = 2
NS = 16
NW = NC * NS
CHUNK = M // NW
UPT = U // NW
C = 16
AUG = 17

_KS0 = np.uint32(0)
_KS1 = np.uint32(42)
_KS2 = np.uint32(0x1BD11BDA ^ 42)
_ROT0 = (13, 15, 26, 6)
_ROT1 = (17, 29, 16, 24)


def _threefry_bits(flat):
    x0 = jnp.zeros_like(flat)
    x1 = flat + _KS1
    ks = (_KS0, _KS1, _KS2)
    for g in range(5):
        for r in (_ROT0 if g % 2 == 0 else _ROT1):
            x0 = x0 + x1
            x1 = (x1 << np.uint32(r)) | (x1 >> np.uint32(32 - r))
            x1 = x1 ^ x0
        x0 = x0 + ks[(g + 1) % 3]
        x1 = x1 + ks[(g + 2) % 3] + np.uint32(g + 1)
    return x0 ^ x1


_MESH = plsc.VectorSubcoreMesh(core_axis_name="c", subcore_axis_name="s")
_PARAMS = pltpu.CompilerParams(needs_layout_passes=False)


@functools.partial(
    pl.kernel,
    mesh=_MESH,
    compiler_params=_PARAMS,
    out_type=(
        jax.ShapeDtypeStruct((NW * U,), jnp.int32),
        jax.ShapeDtypeStruct((NW * U,), jnp.int32),
    ),
    scratch_types=[
        pltpu.VMEM((CHUNK,), jnp.int32),
        pltpu.VMEM((CHUNK,), jnp.int32),
        pltpu.VMEM((CHUNK,), jnp.int32),
        pltpu.VMEM((16 * U,), jnp.int32),
        pltpu.VMEM((16 * U,), jnp.int32),
    ],
)
def _phase1(nn_hbm, ranks_hbm, aug_hbm, tabk_hbm, tabm_hbm,
            mask_v, rank_v, augc_v, tk_v, tm_v):
    wid = lax.axis_index("s") * NC + lax.axis_index("c")
    base = wid * CHUNK
    pltpu.sync_copy(nn_hbm.at[pl.ds(base, CHUNK)], mask_v)
    pltpu.sync_copy(ranks_hbm.at[pl.ds(base, CHUNK)], rank_v)
    pltpu.sync_copy(aug_hbm.at[pl.ds(base, CHUNK)], augc_v)

    lane = lax.iota(jnp.int32, 16)
    neg1 = jnp.full((16,), -1, jnp.int32)

    def init_body(i, _):
        tk_v[pl.ds(i * 16, 16)] = neg1
        return 0

    lax.fori_loop(0, 16 * U // 16, init_body, 0)

    def scatter_body(j, _):
        u16 = mask_v[pl.ds(j * 16, 16)]
        mglob = base + j * 16 + lane
        flat = (u16 * M + mglob).astype(jnp.uint32)
        k16 = (_threefry_bits(flat) >> np.uint32(9)).astype(jnp.int32)
        meta = (mglob << 8) | (rank_v[pl.ds(j * 16, 16)] << 4) \
            | augc_v[pl.ds(j * 16, 16)]
        gidx = lane * U + u16
        cur_k = plsc.load_gather(tk_v, [gidx])
        better = k16 > cur_k
        plsc.store_scatter(tk_v, [gidx], k16, mask=better)
        plsc.store_scatter(tm_v, [gidx], meta, mask=better)
        return 0

    lax.fori_loop(0, CHUNK // 16, scatter_body, 0)

    for s in (8, 4, 2, 1):
        def fold_body(c, _, s=s):
            for l in range(s):
                a = l * U + c * 16
                b = (l + s) * U + c * 16
                ka = tk_v[pl.ds(a, 16)]
                kb = tk_v[pl.ds(b, 16)]
                ma = tm_v[pl.ds(a, 16)]
                mb = tm_v[pl.ds(b, 16)]
                better = (kb > ka) | ((kb == ka) & (mb < ma))
                tk_v[pl.ds(a, 16)] = jnp.where(better, kb, ka)
                tm_v[pl.ds(a, 16)] = jnp.where(better, mb, ma)
            return 0

        lax.fori_loop(0, U // 16, fold_body, 0)

    pltpu.sync_copy(tk_v.at[pl.ds(0, U)], tabk_hbm.at[pl.ds(wid * U, U)])
    pltpu.sync_copy(tm_v.at[pl.ds(0, U)], tabm_hbm.at[pl.ds(wid * U, U)])


@functools.partial(
    pl.kernel,
    mesh=_MESH,
    compiler_params=_PARAMS,
    out_type=(
        jax.ShapeDtypeStruct((U,), jnp.int32),
        jax.ShapeDtypeStruct((U,), jnp.int32),
        jax.ShapeDtypeStruct((U * C,), jnp.float32),
    ),
    scratch_types=[
        pltpu.VMEM((NW * UPT,), jnp.int32),
        pltpu.VMEM((NW * UPT,), jnp.int32),
        pltpu.VMEM((UPT,), jnp.int32),
        pltpu.VMEM((UPT,), jnp.int32),
        pltpu.VMEM((UPT,), jnp.int32),
        pltpu.VMEM((UPT, AUG, C), jnp.float32),
        pltpu.SemaphoreType.DMA,
    ],
)
def _phase2(tabk_hbm, tabm_hbm, ex_hbm, tea_hbm,
            out_idx_hbm, out_rank_hbm, out_tea_hbm,
            kt_v, mt_v, sel_v, rnk_v, ex_v, blk_v, sem):
    wid = lax.axis_index("s") * NC + lax.axis_index("c")
    ubase = wid * UPT
    copies = []
    for src in range(NW):
        copies.append(pltpu.async_copy(
            tabk_hbm.at[pl.ds(src * U + ubase, UPT)],
            kt_v.at[pl.ds(src * UPT, UPT)], sem))
        copies.append(pltpu.async_copy(
            tabm_hbm.at[pl.ds(src * U + ubase, UPT)],
            mt_v.at[pl.ds(src * UPT, UPT)], sem))
    for cp in copies:
        cp.wait()
    pltpu.sync_copy(ex_hbm.at[pl.ds(ubase, UPT)], ex_v)

    augs = []
    for h in range(UPT // 16):
        acck = jnp.full((16,), -1, jnp.int32)
        accm = jnp.full((16,), 0, jnp.int32)
        for src in range(NW):
            k = kt_v[pl.ds(src * UPT + h * 16, 16)]
            m = mt_v[pl.ds(src * UPT + h * 16, 16)]
            better = k > acck
            acck = jnp.where(better, k, acck)
            accm = jnp.where(better, m, accm)
        sel_v[pl.ds(h * 16, 16)] = accm >> 8
        rnk_v[pl.ds(h * 16, 16)] = (accm >> 4) & 15
        augs.append(accm & 15)

    exv = [ex_v[pl.ds(0, 16)], ex_v[pl.ds(16, 16)]]
    blk_cps = []
    for i in range(UPT):
        e_i = exv[i // 16][i % 16]
        blk_cps.append(pltpu.async_copy(tea_hbm.at[e_i], blk_v.at[i], sem))
    for cp in blk_cps:
        cp.wait()

    row_cps = []
    for i in range(UPT):
        aug_i = augs[i // 16][i % 16]
        row_cps.append(pltpu.async_copy(
            blk_v.at[i, aug_i + 1],
            out_tea_hbm.at[pl.ds((ubase + i) * C, C)], sem))
    for cp in row_cps:
        cp.wait()

    pltpu.sync_copy(sel_v, out_idx_hbm.at[pl.ds(ubase, UPT)])
    pltpu.sync_copy(rnk_v, out_rank_hbm.at[pl.ds(ubase, UPT)])


def kernel(stu_logits, teacher_logits, augment_rank, nn_mask, example_indices,
           augmented_indices, nn_ranks):
    del stu_logits, augment_rank
    tabk, tabm = _phase1(nn_mask, nn_ranks, augmented_indices)
    sel_idx, sel_rank, sel_tea = _phase2(
        tabk, tabm, example_indices, teacher_logits)
    return sel_idx, sel_rank, sel_tea.reshape(U, C)


# speedup vs baseline: 14.7455x; 14.7455x over previous
import functools

import numpy as np
import jax
import jax.numpy as jnp
from jax import lax
from jax.experimental import pallas as pl
from jax.experimental.pallas import tpu as pltpu
from jax.experimental.pallas import tpu_sc as plsc

M = 16384
U = 1024
NC = 2
NS = 16
NW = NC * NS
CHUNK = M // NW
UPT = U // NW
C = 16
AUG = 17

_KS0 = np.uint32(0)
_KS1 = np.uint32(42)
_KS2 = np.uint32(0x1BD11BDA ^ 42)
_ROT0 = (13, 15, 26, 6)
_ROT1 = (17, 29, 16, 24)


def _threefry_bits(flat):
    x0 = jnp.zeros_like(flat)
    x1 = flat + _KS1
    ks = (_KS0, _KS1, _KS2)
    for g in range(5):
        for r in (_ROT0 if g % 2 == 0 else _ROT1):
            x0 = x0 + x1
            x1 = (x1 << np.uint32(r)) | (x1 >> np.uint32(32 - r))
            x1 = x1 ^ x0
        x0 = x0 + ks[(g + 1) % 3]
        x1 = x1 + ks[(g + 2) % 3] + np.uint32(g + 1)
    return x0 ^ x1


_MESH = plsc.VectorSubcoreMesh(core_axis_name="c", subcore_axis_name="s")
_PARAMS = pltpu.CompilerParams(needs_layout_passes=False)


@functools.partial(
    pl.kernel,
    mesh=_MESH,
    compiler_params=_PARAMS,
    out_type=(
        jax.ShapeDtypeStruct((NW * U,), jnp.int32),
        jax.ShapeDtypeStruct((NW * U,), jnp.int32),
    ),
    scratch_types=[
        pltpu.VMEM((CHUNK,), jnp.int32),
        pltpu.VMEM((CHUNK,), jnp.int32),
        pltpu.VMEM((CHUNK,), jnp.int32),
        pltpu.VMEM((16 * U,), jnp.int32),
        pltpu.VMEM((16 * U,), jnp.int32),
    ],
)
def _phase1(nn_hbm, ranks_hbm, aug_hbm, tabk_hbm, tabm_hbm,
            mask_v, rank_v, augc_v, tk_v, tm_v):
    wid = lax.axis_index("s") * NC + lax.axis_index("c")
    base = wid * CHUNK
    pltpu.sync_copy(nn_hbm.at[pl.ds(base, CHUNK)], mask_v)
    pltpu.sync_copy(ranks_hbm.at[pl.ds(base, CHUNK)], rank_v)
    pltpu.sync_copy(aug_hbm.at[pl.ds(base, CHUNK)], augc_v)

    lane = lax.iota(jnp.int32, 16)
    neg1 = jnp.full((16,), -1, jnp.int32)

    def init_body(i, _):
        tk_v[pl.ds(i * 16, 16)] = neg1
        return 0

    lax.fori_loop(0, 16 * U // 16, init_body, 0)

    def scatter_body(j, _):
        u16 = mask_v[pl.ds(j * 16, 16)]
        mglob = base + j * 16 + lane
        flat = (u16 * M + mglob).astype(jnp.uint32)
        k16 = (_threefry_bits(flat) >> np.uint32(9)).astype(jnp.int32)
        meta = (mglob << 8) | (rank_v[pl.ds(j * 16, 16)] << 4) \
            | augc_v[pl.ds(j * 16, 16)]
        gidx = lane * U + u16
        cur_k = plsc.load_gather(tk_v, [gidx])
        better = k16 > cur_k
        plsc.store_scatter(tk_v, [gidx], k16, mask=better)
        plsc.store_scatter(tm_v, [gidx], meta, mask=better)
        return 0

    lax.fori_loop(0, CHUNK // 16, scatter_body, 0)

    for s in (8, 4, 2, 1):
        def fold_body(c, _, s=s):
            for l in range(s):
                a = l * U + c * 16
                b = (l + s) * U + c * 16
                ka = tk_v[pl.ds(a, 16)]
                kb = tk_v[pl.ds(b, 16)]
                ma = tm_v[pl.ds(a, 16)]
                mb = tm_v[pl.ds(b, 16)]
                better = (kb > ka) | ((kb == ka) & (mb < ma))
                tk_v[pl.ds(a, 16)] = jnp.where(better, kb, ka)
                tm_v[pl.ds(a, 16)] = jnp.where(better, mb, ma)
            return 0

        lax.fori_loop(0, U // 16, fold_body, 0)

    pltpu.sync_copy(tk_v.at[pl.ds(0, U)], tabk_hbm.at[pl.ds(wid * U, U)])
    pltpu.sync_copy(tm_v.at[pl.ds(0, U)], tabm_hbm.at[pl.ds(wid * U, U)])


@functools.partial(
    pl.kernel,
    mesh=_MESH,
    compiler_params=_PARAMS,
    out_type=(
        jax.ShapeDtypeStruct((U,), jnp.int32),
        jax.ShapeDtypeStruct((U,), jnp.int32),
        jax.ShapeDtypeStruct((U * C,), jnp.float32),
    ),
    scratch_types=[
        pltpu.VMEM((NW * UPT,), jnp.int32),
        pltpu.VMEM((NW * UPT,), jnp.int32),
        pltpu.VMEM((UPT,), jnp.int32),
        pltpu.VMEM((UPT,), jnp.int32),
        pltpu.VMEM((UPT,), jnp.int32),
        pltpu.VMEM((UPT, C, 128), jnp.float32),
        pltpu.VMEM((UPT * C,), jnp.float32),
        pltpu.SemaphoreType.DMA,
    ],
)
def _phase2(tabk_hbm, tabm_hbm, ex_hbm, tea_hbm,
            out_idx_hbm, out_rank_hbm, out_tea_hbm,
            kt_v, mt_v, sel_v, rnk_v, ex_v, win_v, row_v, sem):
    wid = lax.axis_index("s") * NC + lax.axis_index("c")
    ubase = wid * UPT
    copies = []
    for src in range(NW):
        copies.append(pltpu.async_copy(
            tabk_hbm.at[pl.ds(src * U + ubase, UPT)],
            kt_v.at[pl.ds(src * UPT, UPT)], sem))
        copies.append(pltpu.async_copy(
            tabm_hbm.at[pl.ds(src * U + ubase, UPT)],
            mt_v.at[pl.ds(src * UPT, UPT)], sem))
    for cp in copies:
        cp.wait()
    pltpu.sync_copy(ex_hbm.at[pl.ds(ubase, UPT)], ex_v)

    augs = []
    for h in range(UPT // 16):
        acck = jnp.full((16,), -1, jnp.int32)
        accm = jnp.full((16,), 0, jnp.int32)
        for src in range(NW):
            k = kt_v[pl.ds(src * UPT + h * 16, 16)]
            m = mt_v[pl.ds(src * UPT + h * 16, 16)]
            better = k > acck
            acck = jnp.where(better, k, acck)
            accm = jnp.where(better, m, accm)
        sel_v[pl.ds(h * 16, 16)] = accm >> 8
        rnk_v[pl.ds(h * 16, 16)] = (accm >> 4) & 15
        augs.append(accm & 15)

    exv = [ex_v[pl.ds(0, 16)], ex_v[pl.ds(16, 16)]]
    ebv = [e & -128 for e in exv]
    eov = [e & 127 for e in exv]
    blk_cps = []
    for i in range(UPT):
        aug_i = augs[i // 16][i % 16]
        eb_i = pl.multiple_of(ebv[i // 16][i % 16], 128)
        blk_cps.append(pltpu.async_copy(
            tea_hbm.at[aug_i + 1, :, pl.ds(eb_i, 128)], win_v.at[i], sem))
    for cp in blk_cps:
        cp.wait()

    lane = lax.iota(jnp.int32, 16)
    zeros = jnp.zeros((16,), jnp.int32)
    for i in range(UPT):
        eo_i = eov[i // 16][i % 16]
        row = plsc.load_gather(win_v, [zeros + i, lane, zeros + eo_i])
        row_v[pl.ds(i * C, C)] = row

    pltpu.sync_copy(row_v, out_tea_hbm.at[pl.ds(ubase * C, UPT * C)])
    pltpu.sync_copy(sel_v, out_idx_hbm.at[pl.ds(ubase, UPT)])
    pltpu.sync_copy(rnk_v, out_rank_hbm.at[pl.ds(ubase, UPT)])


def kernel(stu_logits, teacher_logits, augment_rank, nn_mask, example_indices,
           augmented_indices, nn_ranks):
    del stu_logits, augment_rank
    tabk, tabm = _phase1(nn_mask, nn_ranks, augmented_indices)
    tea_t = jnp.transpose(teacher_logits, (1, 2, 0))
    sel_idx, sel_rank, sel_tea = _phase2(
        tabk, tabm, example_indices, tea_t)
    return sel_idx, sel_rank, sel_tea.reshape(U, C)


# speedup vs baseline: 17.6431x; 1.1965x over previous
import functools

import numpy as np
import jax
import jax.numpy as jnp
from jax import lax
from jax.experimental import pallas as pl
from jax.experimental.pallas import tpu as pltpu
from jax.experimental.pallas import tpu_sc as plsc

M = 16384
U = 1024
NC = 2
NS = 16
HALF = U // NC
EPT = M // NS
UPT = HALF // NS
C = 16

_KS0 = np.uint32(0)
_KS1 = np.uint32(42)
_KS2 = np.uint32(0x1BD11BDA ^ 42)
_ROT0 = (13, 15, 26, 6)
_ROT1 = (17, 29, 16, 24)


def _threefry_bits(flat):
    x0 = jnp.zeros_like(flat)
    x1 = flat + _KS1
    ks = (_KS0, _KS1, _KS2)
    for g in range(5):
        for r in (_ROT0 if g % 2 == 0 else _ROT1):
            x0 = x0 + x1
            x1 = (x1 << np.uint32(r)) | (x1 >> np.uint32(32 - r))
            x1 = x1 ^ x0
        x0 = x0 + ks[(g + 1) % 3]
        x1 = x1 + ks[(g + 2) % 3] + np.uint32(g + 1)
    return x0 ^ x1


_MESH = plsc.VectorSubcoreMesh(core_axis_name="c", subcore_axis_name="s")
_PARAMS = pltpu.CompilerParams(needs_layout_passes=False)


@functools.partial(
    pl.kernel,
    mesh=_MESH,
    compiler_params=_PARAMS,
    out_type=(
        jax.ShapeDtypeStruct((U,), jnp.int32),
        jax.ShapeDtypeStruct((U,), jnp.int32),
        jax.ShapeDtypeStruct((U * C,), jnp.float32),
    ),
    scratch_types=[
        pltpu.VMEM((EPT,), jnp.int32),
        pltpu.VMEM((EPT,), jnp.int32),
        pltpu.VMEM((EPT,), jnp.int32),
        pltpu.VMEM((16 * HALF,), jnp.int32),
        pltpu.VMEM((16 * HALF,), jnp.int32),
        pltpu.VMEM((NS * UPT,), jnp.int32),
        pltpu.VMEM((NS * UPT,), jnp.int32),
        pltpu.VMEM((UPT,), jnp.int32),
        pltpu.VMEM((UPT,), jnp.int32),
        pltpu.VMEM((UPT,), jnp.int32),
        pltpu.VMEM((UPT, C, 128), jnp.float32),
        pltpu.VMEM((UPT * C,), jnp.float32),
        pltpu.VMEM_SHARED((NS, HALF), jnp.int32),
        pltpu.VMEM_SHARED((NS, HALF), jnp.int32),
        pltpu.SemaphoreType.DMA,
    ],
)
def _fused(nn_hbm, ranks_hbm, aug_hbm, ex_hbm, tea_hbm,
           out_idx_hbm, out_rank_hbm, out_tea_hbm,
           mask_v, rank_v, augc_v, tk_v, tm_v, kt_v, mt_v,
           sel_v, rnk_v, ex_v, win_v, row_v, shk_s, shm_s, sem):
    cid = lax.axis_index("c")
    sid = lax.axis_index("s")
    base = sid * EPT
    pltpu.sync_copy(nn_hbm.at[pl.ds(base, EPT)], mask_v)
    pltpu.sync_copy(ranks_hbm.at[pl.ds(base, EPT)], rank_v)
    pltpu.sync_copy(aug_hbm.at[pl.ds(base, EPT)], augc_v)

    lane = lax.iota(jnp.int32, 16)
    neg1 = jnp.full((16,), -1, jnp.int32)

    def init_body(i, _):
        tk_v[pl.ds(i * 16, 16)] = neg1
        return 0

    lax.fori_loop(0, 16 * HALF // 16, init_body, 0)

    def scatter_body(j, _):
        u16 = mask_v[pl.ds(j * 16, 16)]
        mglob = base + j * 16 + lane
        flat = (u16 * M + mglob).astype(jnp.uint32)
        k16 = (_threefry_bits(flat) >> np.uint32(9)).astype(jnp.int32)
        meta = (mglob << 8) | (rank_v[pl.ds(j * 16, 16)] << 4) \
            | augc_v[pl.ds(j * 16, 16)]
        inhalf = (u16 >> 9) == cid
        gidx = lane * HALF + (u16 & (HALF - 1))
        cur_k = plsc.load_gather(tk_v, [gidx])
        better = inhalf & (k16 > cur_k)
        plsc.store_scatter(tk_v, [gidx], k16, mask=better)
        plsc.store_scatter(tm_v, [gidx], meta, mask=better)
        return 0

    lax.fori_loop(0, EPT // 16, scatter_body, 0)

    for s in (8, 4, 2, 1):
        def fold_body(c, _, s=s):
            for l in range(s):
                a = l * HALF + c * 16
                b = (l + s) * HALF + c * 16
                ka = tk_v[pl.ds(a, 16)]
                kb = tk_v[pl.ds(b, 16)]
                ma = tm_v[pl.ds(a, 16)]
                mb = tm_v[pl.ds(b, 16)]
                better = (kb > ka) | ((kb == ka) & (mb < ma))
                tk_v[pl.ds(a, 16)] = jnp.where(better, kb, ka)
                tm_v[pl.ds(a, 16)] = jnp.where(better, mb, ma)
            return 0

        lax.fori_loop(0, HALF // 16, fold_body, 0)

    pltpu.sync_copy(tk_v.at[pl.ds(0, HALF)], shk_s.at[sid])
    pltpu.sync_copy(tm_v.at[pl.ds(0, HALF)], shm_s.at[sid])
    plsc.subcore_barrier()

    lubase = sid * UPT
    tab_cps = []
    for src in range(NS):
        tab_cps.append(pltpu.async_copy(
            shk_s.at[src, pl.ds(lubase, UPT)],
            kt_v.at[pl.ds(src * UPT, UPT)], sem))
        tab_cps.append(pltpu.async_copy(
            shm_s.at[src, pl.ds(lubase, UPT)],
            mt_v.at[pl.ds(src * UPT, UPT)], sem))
    for cp in tab_cps:
        cp.wait()

    ubase = cid * HALF + lubase
    pltpu.sync_copy(ex_hbm.at[pl.ds(ubase, UPT)], ex_v)

    augs = []
    for h in range(UPT // 16):
        acck = jnp.full((16,), -1, jnp.int32)
        accm = jnp.full((16,), 0, jnp.int32)
        for src in range(NS):
            k = kt_v[pl.ds(src * UPT + h * 16, 16)]
            m = mt_v[pl.ds(src * UPT + h * 16, 16)]
            better = k > acck
            acck = jnp.where(better, k, acck)
            accm = jnp.where(better, m, accm)
        sel_v[pl.ds(h * 16, 16)] = accm >> 8
        rnk_v[pl.ds(h * 16, 16)] = (accm >> 4) & 15
        augs.append(accm & 15)

    exv = [ex_v[pl.ds(0, 16)], ex_v[pl.ds(16, 16)]]
    ebv = [e & -128 for e in exv]
    eov = [e & 127 for e in exv]
    blk_cps = []
    for i in range(UPT):
        aug_i = augs[i // 16][i % 16]
        eb_i = pl.multiple_of(ebv[i // 16][i % 16], 128)
        blk_cps.append(pltpu.async_copy(
            tea_hbm.at[aug_i + 1, :, pl.ds(eb_i, 128)], win_v.at[i], sem))
    for cp in blk_cps:
        cp.wait()

    zeros = jnp.zeros((16,), jnp.int32)
    for i in range(UPT):
        eo_i = eov[i // 16][i % 16]
        row = plsc.load_gather(win_v, [zeros + i, lane, zeros + eo_i])
        row_v[pl.ds(i * C, C)] = row

    pltpu.sync_copy(row_v, out_tea_hbm.at[pl.ds(ubase * C, UPT * C)])
    pltpu.sync_copy(sel_v, out_idx_hbm.at[pl.ds(ubase, UPT)])
    pltpu.sync_copy(rnk_v, out_rank_hbm.at[pl.ds(ubase, UPT)])


def kernel(stu_logits, teacher_logits, augment_rank, nn_mask, example_indices,
           augmented_indices, nn_ranks):
    del stu_logits, augment_rank
    tea_t = jnp.transpose(teacher_logits, (1, 2, 0))
    sel_idx, sel_rank, sel_tea = _fused(
        nn_mask, nn_ranks, augmented_indices, example_indices, tea_t)
    return sel_idx, sel_rank, sel_tea.reshape(U, C)


# speedup vs baseline: 18.4473x; 1.0456x over previous
import functools

import numpy as np
import jax
import jax.numpy as jnp
from jax import lax
from jax.experimental import pallas as pl
from jax.experimental.pallas import tpu as pltpu
from jax.experimental.pallas import tpu_sc as plsc

M = 16384
U = 1024
NC = 2
NS = 16
HALF = U // NC
EPT = M // NS
UPT = HALF // NS
C = 16

_KS0 = np.uint32(0)
_KS1 = np.uint32(42)
_KS2 = np.uint32(0x1BD11BDA ^ 42)
_ROT0 = (13, 15, 26, 6)
_ROT1 = (17, 29, 16, 24)


def _threefry_bits(flat):
    x0 = jnp.zeros_like(flat)
    x1 = flat + _KS1
    ks = (_KS0, _KS1, _KS2)
    for g in range(5):
        for r in (_ROT0 if g % 2 == 0 else _ROT1):
            x0 = x0 + x1
            x1 = (x1 << np.uint32(r)) | (x1 >> np.uint32(32 - r))
            x1 = x1 ^ x0
        x0 = x0 + ks[(g + 1) % 3]
        x1 = x1 + ks[(g + 2) % 3] + np.uint32(g + 1)
    return x0 ^ x1


_MESH = plsc.VectorSubcoreMesh(core_axis_name="c", subcore_axis_name="s")
_PARAMS = pltpu.CompilerParams(
    needs_layout_passes=False,
    disable_bounds_checks=True,
    disable_semaphore_checks=True,
)


@functools.partial(
    pl.kernel,
    mesh=_MESH,
    compiler_params=_PARAMS,
    out_type=(
        jax.ShapeDtypeStruct((U,), jnp.int32),
        jax.ShapeDtypeStruct((U,), jnp.int32),
        jax.ShapeDtypeStruct((U * C,), jnp.float32),
    ),
    scratch_types=[
        pltpu.VMEM((EPT,), jnp.int32),
        pltpu.VMEM((EPT,), jnp.int32),
        pltpu.VMEM((EPT,), jnp.int32),
        pltpu.VMEM((16 * HALF,), jnp.int32),
        pltpu.VMEM((16 * HALF,), jnp.int32),
        pltpu.VMEM((NS * UPT,), jnp.int32),
        pltpu.VMEM((NS * UPT,), jnp.int32),
        pltpu.VMEM((UPT,), jnp.int32),
        pltpu.VMEM((UPT,), jnp.int32),
        pltpu.VMEM((UPT,), jnp.int32),
        pltpu.VMEM((UPT, C, 128), jnp.float32),
        pltpu.VMEM((UPT * C,), jnp.float32),
        pltpu.VMEM_SHARED((NS, HALF), jnp.int32),
        pltpu.VMEM_SHARED((NS, HALF), jnp.int32),
        pltpu.SemaphoreType.DMA,
    ],
)
def _fused(nn_hbm, ranks_hbm, aug_hbm, ex_hbm, tea_hbm,
           out_idx_hbm, out_rank_hbm, out_tea_hbm,
           mask_v, rank_v, augc_v, tk_v, tm_v, kt_v, mt_v,
           sel_v, rnk_v, ex_v, win_v, row_v, shk_s, shm_s, sem):
    cid = lax.axis_index("c")
    sid = lax.axis_index("s")
    base = sid * EPT
    pltpu.sync_copy(nn_hbm.at[pl.ds(base, EPT)], mask_v)
    pltpu.sync_copy(ranks_hbm.at[pl.ds(base, EPT)], rank_v)
    pltpu.sync_copy(aug_hbm.at[pl.ds(base, EPT)], augc_v)

    lane = lax.iota(jnp.int32, 16)
    neg1 = jnp.full((16,), -1, jnp.int32)

    def init_body(i, _):
        for r in range(8):
            tk_v[pl.ds(i * 128 + r * 16, 16)] = neg1
        return 0

    lax.fori_loop(0, 16 * HALF // 128, init_body, 0)

    def scatter_body(j, _):
        u16 = mask_v[pl.ds(j * 16, 16)]
        mglob = base + j * 16 + lane
        flat = (u16 * M + mglob).astype(jnp.uint32)
        k16 = (_threefry_bits(flat) >> np.uint32(9)).astype(jnp.int32)
        meta = (mglob << 8) | (rank_v[pl.ds(j * 16, 16)] << 4) \
            | augc_v[pl.ds(j * 16, 16)]
        inhalf = (u16 >> 9) == cid
        gidx = lane * HALF + (u16 & (HALF - 1))
        cur_k = plsc.load_gather(tk_v, [gidx])
        better = inhalf & (k16 > cur_k)
        plsc.store_scatter(tk_v, [gidx], k16, mask=better)
        plsc.store_scatter(tm_v, [gidx], meta, mask=better)
        return 0

    lax.fori_loop(0, EPT // 16, scatter_body, 0)

    for s in (8, 4, 2, 1):
        unroll = 4 if s <= 2 else 1

        def fold_body(c, _, s=s, unroll=unroll):
            for w in range(unroll):
                for l in range(s):
                    a = l * HALF + (c * unroll + w) * 16
                    b = (l + s) * HALF + (c * unroll + w) * 16
                    ka = tk_v[pl.ds(a, 16)]
                    kb = tk_v[pl.ds(b, 16)]
                    ma = tm_v[pl.ds(a, 16)]
                    mb = tm_v[pl.ds(b, 16)]
                    better = (kb > ka) | ((kb == ka) & (mb < ma))
                    tk_v[pl.ds(a, 16)] = jnp.where(better, kb, ka)
                    tm_v[pl.ds(a, 16)] = jnp.where(better, mb, ma)
            return 0

        lax.fori_loop(0, HALF // 16 // unroll, fold_body, 0)

    pltpu.sync_copy(tk_v.at[pl.ds(0, HALF)], shk_s.at[sid])
    pltpu.sync_copy(tm_v.at[pl.ds(0, HALF)], shm_s.at[sid])
    plsc.subcore_barrier()

    lubase = sid * UPT
    tab_cps = []
    for src in range(NS):
        tab_cps.append(pltpu.async_copy(
            shk_s.at[src, pl.ds(lubase, UPT)],
            kt_v.at[pl.ds(src * UPT, UPT)], sem))
        tab_cps.append(pltpu.async_copy(
            shm_s.at[src, pl.ds(lubase, UPT)],
            mt_v.at[pl.ds(src * UPT, UPT)], sem))
    for cp in tab_cps:
        cp.wait()

    ubase = cid * HALF + lubase
    pltpu.sync_copy(ex_hbm.at[pl.ds(ubase, UPT)], ex_v)

    augs = []
    for h in range(UPT // 16):
        acck = jnp.full((16,), -1, jnp.int32)
        accm = jnp.full((16,), 0, jnp.int32)
        for src in range(NS):
            k = kt_v[pl.ds(src * UPT + h * 16, 16)]
            m = mt_v[pl.ds(src * UPT + h * 16, 16)]
            better = k > acck
            acck = jnp.where(better, k, acck)
            accm = jnp.where(better, m, accm)
        sel_v[pl.ds(h * 16, 16)] = accm >> 8
        rnk_v[pl.ds(h * 16, 16)] = (accm >> 4) & 15
        augs.append(accm & 15)

    exv = [ex_v[pl.ds(0, 16)], ex_v[pl.ds(16, 16)]]
    ebv = [e & -128 for e in exv]
    eov = [e & 127 for e in exv]
    blk_cps = []
    for i in range(UPT):
        aug_i = augs[i // 16][i % 16]
        eb_i = pl.multiple_of(ebv[i // 16][i % 16], 128)
        blk_cps.append(pltpu.async_copy(
            tea_hbm.at[aug_i + 1, :, pl.ds(eb_i, 128)], win_v.at[i], sem))
    for cp in blk_cps:
        cp.wait()

    zeros = jnp.zeros((16,), jnp.int32)
    for i in range(UPT):
        eo_i = eov[i // 16][i % 16]
        row = plsc.load_gather(win_v, [zeros + i, lane, zeros + eo_i])
        row_v[pl.ds(i * C, C)] = row

    pltpu.sync_copy(row_v, out_tea_hbm.at[pl.ds(ubase * C, UPT * C)])
    pltpu.sync_copy(sel_v, out_idx_hbm.at[pl.ds(ubase, UPT)])
    pltpu.sync_copy(rnk_v, out_rank_hbm.at[pl.ds(ubase, UPT)])


def kernel(stu_logits, teacher_logits, augment_rank, nn_mask, example_indices,
           augmented_indices, nn_ranks):
    del stu_logits, augment_rank
    tea_t = jnp.transpose(teacher_logits, (1, 2, 0))
    sel_idx, sel_rank, sel_tea = _fused(
        nn_mask, nn_ranks, augmented_indices, example_indices, tea_t)
    return sel_idx, sel_rank, sel_tea.reshape(U, C)


# speedup vs baseline: 19.6835x; 1.0670x over previous
import functools

import numpy as np
import jax
import jax.numpy as jnp
from jax import lax
from jax.experimental import pallas as pl
from jax.experimental.pallas import tpu as pltpu
from jax.experimental.pallas import tpu_sc as plsc

M = 16384
U = 1024
NC = 2
NS = 16
HALF = U // NC
EPT = M // NS
UPT = HALF // NS
C = 16

_KS0 = np.uint32(0)
_KS1 = np.uint32(42)
_KS2 = np.uint32(0x1BD11BDA ^ 42)
_ROT0 = (13, 15, 26, 6)
_ROT1 = (17, 29, 16, 24)


def _threefry_bits(flat):
    x0 = jnp.zeros_like(flat)
    x1 = flat + _KS1
    ks = (_KS0, _KS1, _KS2)
    for g in range(5):
        for r in (_ROT0 if g % 2 == 0 else _ROT1):
            x0 = x0 + x1
            x1 = (x1 << np.uint32(r)) | (x1 >> np.uint32(32 - r))
            x1 = x1 ^ x0
        x0 = x0 + ks[(g + 1) % 3]
        x1 = x1 + ks[(g + 2) % 3] + np.uint32(g + 1)
    return x0 ^ x1


_MESH = plsc.VectorSubcoreMesh(core_axis_name="c", subcore_axis_name="s")
_PARAMS = pltpu.CompilerParams(
    needs_layout_passes=False,
    disable_bounds_checks=True,
    disable_semaphore_checks=True,
)


@functools.partial(
    pl.kernel,
    mesh=_MESH,
    compiler_params=_PARAMS,
    out_type=(
        jax.ShapeDtypeStruct((U,), jnp.int32),
        jax.ShapeDtypeStruct((U,), jnp.int32),
        jax.ShapeDtypeStruct((U * C,), jnp.float32),
    ),
    scratch_types=[
        pltpu.VMEM((EPT,), jnp.int32),
        pltpu.VMEM((EPT,), jnp.int32),
        pltpu.VMEM((EPT,), jnp.int32),
        pltpu.VMEM((16 * HALF,), jnp.int32),
        pltpu.VMEM((16 * HALF,), jnp.int32),
        pltpu.VMEM((NS * UPT,), jnp.int32),
        pltpu.VMEM((NS * UPT,), jnp.int32),
        pltpu.VMEM((UPT,), jnp.int32),
        pltpu.VMEM((UPT,), jnp.int32),
        pltpu.VMEM((UPT,), jnp.int32),
        pltpu.VMEM((UPT, C, 128), jnp.float32),
        pltpu.VMEM((C * UPT,), jnp.float32),
        pltpu.VMEM_SHARED((NS, HALF), jnp.int32),
        pltpu.VMEM_SHARED((NS, HALF), jnp.int32),
        pltpu.SemaphoreType.DMA,
    ],
)
def _fused(nn_hbm, ranks_hbm, aug_hbm, ex_hbm, tea_hbm,
           out_idx_hbm, out_rank_hbm, out_tea_hbm,
           mask_v, rank_v, augc_v, tk_v, tm_v, kt_v, mt_v,
           sel_v, rnk_v, ex_v, win_v, row_v, shk_s, shm_s, sem):
    cid = lax.axis_index("c")
    sid = lax.axis_index("s")
    base = sid * EPT
    lubase = sid * UPT
    ubase = cid * HALF + lubase
    in_cps = [
        pltpu.async_copy(nn_hbm.at[pl.ds(base, EPT)], mask_v, sem),
        pltpu.async_copy(ranks_hbm.at[pl.ds(base, EPT)], rank_v, sem),
        pltpu.async_copy(aug_hbm.at[pl.ds(base, EPT)], augc_v, sem),
        pltpu.async_copy(ex_hbm.at[pl.ds(ubase, UPT)], ex_v, sem),
    ]

    lane = lax.iota(jnp.int32, 16)
    neg1 = jnp.full((16,), -1, jnp.int32)

    def init_body(i, _):
        for r in range(8):
            tk_v[pl.ds(i * 128 + r * 16, 16)] = neg1
        return 0

    lax.fori_loop(0, 16 * HALF // 128, init_body, 0)
    for cp in in_cps:
        cp.wait()

    def scatter_body(j, _):
        u16 = mask_v[pl.ds(j * 16, 16)]
        mglob = base + j * 16 + lane
        flat = (u16 * M + mglob).astype(jnp.uint32)
        k16 = (_threefry_bits(flat) >> np.uint32(9)).astype(jnp.int32)
        meta = (mglob << 8) | (rank_v[pl.ds(j * 16, 16)] << 4) \
            | augc_v[pl.ds(j * 16, 16)]
        inhalf = (u16 >> 9) == cid
        gidx = lane * HALF + (u16 & (HALF - 1))
        cur_k = plsc.load_gather(tk_v, [gidx])
        better = inhalf & (k16 > cur_k)
        plsc.store_scatter(tk_v, [gidx], k16, mask=better)
        plsc.store_scatter(tm_v, [gidx], meta, mask=better)
        return 0

    lax.fori_loop(0, EPT // 16, scatter_body, 0)

    for s in (8, 4, 2, 1):
        unroll = 4 if s <= 2 else 1

        def fold_body(c, _, s=s, unroll=unroll):
            for w in range(unroll):
                for l in range(s):
                    a = l * HALF + (c * unroll + w) * 16
                    b = (l + s) * HALF + (c * unroll + w) * 16
                    ka = tk_v[pl.ds(a, 16)]
                    kb = tk_v[pl.ds(b, 16)]
                    ma = tm_v[pl.ds(a, 16)]
                    mb = tm_v[pl.ds(b, 16)]
                    better = (kb > ka) | ((kb == ka) & (mb < ma))
                    tk_v[pl.ds(a, 16)] = jnp.where(better, kb, ka)
                    tm_v[pl.ds(a, 16)] = jnp.where(better, mb, ma)
            return 0

        lax.fori_loop(0, HALF // 16 // unroll, fold_body, 0)

    pltpu.sync_copy(tk_v.at[pl.ds(0, HALF)], shk_s.at[sid])
    pltpu.sync_copy(tm_v.at[pl.ds(0, HALF)], shm_s.at[sid])
    plsc.subcore_barrier()

    tab_cps = []
    for src in range(NS):
        tab_cps.append(pltpu.async_copy(
            shk_s.at[src, pl.ds(lubase, UPT)],
            kt_v.at[pl.ds(src * UPT, UPT)], sem))
        tab_cps.append(pltpu.async_copy(
            shm_s.at[src, pl.ds(lubase, UPT)],
            mt_v.at[pl.ds(src * UPT, UPT)], sem))
    for cp in tab_cps:
        cp.wait()

    augs = []
    for h in range(UPT // 16):
        acck = jnp.full((16,), -1, jnp.int32)
        accm = jnp.full((16,), 0, jnp.int32)
        for src in range(NS):
            k = kt_v[pl.ds(src * UPT + h * 16, 16)]
            m = mt_v[pl.ds(src * UPT + h * 16, 16)]
            better = k > acck
            acck = jnp.where(better, k, acck)
            accm = jnp.where(better, m, accm)
        sel_v[pl.ds(h * 16, 16)] = accm >> 8
        rnk_v[pl.ds(h * 16, 16)] = (accm >> 4) & 15
        augs.append(accm & 15)

    exv = [ex_v[pl.ds(0, 16)], ex_v[pl.ds(16, 16)]]
    ebv = [e & -128 for e in exv]
    eov = [e & 127 for e in exv]
    blk_cps = []
    for i in range(UPT):
        aug_i = augs[i // 16][i % 16]
        eb_i = pl.multiple_of(ebv[i // 16][i % 16], 128)
        blk_cps.append(pltpu.async_copy(
            tea_hbm.at[aug_i + 1, :, pl.ds(eb_i, 128)], win_v.at[i], sem))
    for cp in blk_cps:
        cp.wait()

    zeros = jnp.zeros((16,), jnp.int32)
    for c in range(C):
        for h in range(UPT // 16):
            vals = plsc.load_gather(win_v, [lane + h * 16, zeros + c, eov[h]])
            row_v[pl.ds(c * UPT + h * 16, 16)] = vals

    out_cps = []
    for c in range(C):
        out_cps.append(pltpu.async_copy(
            row_v.at[pl.ds(c * UPT, UPT)],
            out_tea_hbm.at[pl.ds(c * U + ubase, UPT)], sem))
    pltpu.sync_copy(sel_v, out_idx_hbm.at[pl.ds(ubase, UPT)])
    pltpu.sync_copy(rnk_v, out_rank_hbm.at[pl.ds(ubase, UPT)])
    for cp in out_cps:
        cp.wait()


def kernel(stu_logits, teacher_logits, augment_rank, nn_mask, example_indices,
           augmented_indices, nn_ranks):
    del stu_logits, augment_rank
    tea_t = jnp.transpose(teacher_logits, (1, 2, 0))
    sel_idx, sel_rank, sel_tea = _fused(
        nn_mask, nn_ranks, augmented_indices, example_indices, tea_t)
    return sel_idx, sel_rank, sel_tea.reshape(C, U).T


# speedup vs baseline: 19.8371x; 1.0078x over previous
import functools

import numpy as np
import jax
import jax.numpy as jnp
from jax import lax
from jax.experimental import pallas as pl
from jax.experimental.pallas import tpu as pltpu
from jax.experimental.pallas import tpu_sc as plsc

M = 16384
U = 1024
NC = 2
NS = 16
HALF = U // NC
EPT = M // NS
UPT = HALF // NS
C = 16

_KS0 = np.uint32(0)
_KS1 = np.uint32(42)
_KS2 = np.uint32(0x1BD11BDA ^ 42)
_ROT0 = (13, 15, 26, 6)
_ROT1 = (17, 29, 16, 24)


def _threefry_bits(flat):
    x0 = jnp.zeros_like(flat)
    x1 = flat + _KS1
    ks = (_KS0, _KS1, _KS2)
    for g in range(5):
        for r in (_ROT0 if g % 2 == 0 else _ROT1):
            x0 = x0 + x1
            x1 = (x1 << np.uint32(r)) | (x1 >> np.uint32(32 - r))
            x1 = x1 ^ x0
        x0 = x0 + ks[(g + 1) % 3]
        x1 = x1 + ks[(g + 2) % 3] + np.uint32(g + 1)
    return x0 ^ x1


_MESH = plsc.VectorSubcoreMesh(core_axis_name="c", subcore_axis_name="s")
_PARAMS = pltpu.CompilerParams(
    needs_layout_passes=False,
    disable_bounds_checks=True,
    disable_semaphore_checks=True,
)


@functools.partial(
    pl.kernel,
    mesh=_MESH,
    compiler_params=_PARAMS,
    out_type=(
        jax.ShapeDtypeStruct((U,), jnp.int32),
        jax.ShapeDtypeStruct((U,), jnp.int32),
        jax.ShapeDtypeStruct((U * C,), jnp.float32),
    ),
    scratch_types=[
        pltpu.VMEM((EPT,), jnp.int32),
        pltpu.VMEM((EPT,), jnp.int32),
        pltpu.VMEM((EPT,), jnp.int32),
        pltpu.VMEM((16 * HALF,), jnp.int32),
        pltpu.VMEM((16 * HALF,), jnp.int32),
        pltpu.VMEM((NS * UPT,), jnp.int32),
        pltpu.VMEM((NS * UPT,), jnp.int32),
        pltpu.VMEM((UPT,), jnp.int32),
        pltpu.VMEM((UPT,), jnp.int32),
        pltpu.VMEM((UPT,), jnp.int32),
        pltpu.VMEM((UPT, C, 128), jnp.float32),
        pltpu.VMEM((C * UPT,), jnp.float32),
        pltpu.VMEM_SHARED((NS, HALF), jnp.int32),
        pltpu.VMEM_SHARED((NS, HALF), jnp.int32),
        pltpu.SemaphoreType.DMA,
    ],
)
def _fused(nn_hbm, ranks_hbm, aug_hbm, ex_hbm, tea_hbm,
           out_idx_hbm, out_rank_hbm, out_tea_hbm,
           mask_v, rank_v, augc_v, tk_v, tm_v, kt_v, mt_v,
           sel_v, rnk_v, ex_v, win_v, row_v, shk_s, shm_s, sem):
    cid = lax.axis_index("c")
    sid = lax.axis_index("s")
    base = sid * EPT
    lubase = sid * UPT
    ubase = cid * HALF + lubase
    in_cps = [
        pltpu.async_copy(nn_hbm.at[pl.ds(base, EPT)], mask_v, sem),
        pltpu.async_copy(ranks_hbm.at[pl.ds(base, EPT)], rank_v, sem),
        pltpu.async_copy(aug_hbm.at[pl.ds(base, EPT)], augc_v, sem),
        pltpu.async_copy(ex_hbm.at[pl.ds(ubase, UPT)], ex_v, sem),
    ]

    lane = lax.iota(jnp.int32, 16)
    neg1 = jnp.full((16,), -1, jnp.int32)

    def init_body(i, _):
        for r in range(8):
            tk_v[pl.ds(i * 128 + r * 16, 16)] = neg1
        return 0

    lax.fori_loop(0, 16 * HALF // 128, init_body, 0)
    for cp in in_cps:
        cp.wait()

    def scatter_body(j, _):
        u16 = mask_v[pl.ds(j * 16, 16)]
        mglob = base + j * 16 + lane
        flat = (u16 * M + mglob).astype(jnp.uint32)
        k16 = (_threefry_bits(flat) >> np.uint32(9)).astype(jnp.int32)
        meta = (mglob << 8) | (rank_v[pl.ds(j * 16, 16)] << 4) \
            | augc_v[pl.ds(j * 16, 16)]
        inhalf = (u16 >> 9) == cid
        gidx = lane * HALF + (u16 & (HALF - 1))
        cur_k = plsc.load_gather(tk_v, [gidx])
        better = inhalf & (k16 > cur_k)
        plsc.store_scatter(tk_v, [gidx], k16, mask=better)
        plsc.store_scatter(tm_v, [gidx], meta, mask=better)
        return 0

    lax.fori_loop(0, EPT // 16, scatter_body, 0)

    for s in (8, 4, 2, 1):
        unroll = 4 if s <= 2 else 1

        def fold_body(c, _, s=s, unroll=unroll):
            for w in range(unroll):
                for l in range(s):
                    a = l * HALF + (c * unroll + w) * 16
                    b = (l + s) * HALF + (c * unroll + w) * 16
                    ka = tk_v[pl.ds(a, 16)]
                    kb = tk_v[pl.ds(b, 16)]
                    ma = tm_v[pl.ds(a, 16)]
                    mb = tm_v[pl.ds(b, 16)]
                    better = (kb > ka) | ((kb == ka) & (mb < ma))
                    tk_v[pl.ds(a, 16)] = jnp.where(better, kb, ka)
                    tm_v[pl.ds(a, 16)] = jnp.where(better, mb, ma)
            return 0

        lax.fori_loop(0, HALF // 16 // unroll, fold_body, 0)

    pltpu.sync_copy(tk_v.at[pl.ds(0, HALF)], shk_s.at[sid])
    pltpu.sync_copy(tm_v.at[pl.ds(0, HALF)], shm_s.at[sid])
    plsc.subcore_barrier()

    tab_cps = []
    for src in range(NS):
        tab_cps.append(pltpu.async_copy(
            shk_s.at[src, pl.ds(lubase, UPT)],
            kt_v.at[pl.ds(src * UPT, UPT)], sem))
        tab_cps.append(pltpu.async_copy(
            shm_s.at[src, pl.ds(lubase, UPT)],
            mt_v.at[pl.ds(src * UPT, UPT)], sem))
    for cp in tab_cps:
        cp.wait()

    augs = []
    for h in range(UPT // 16):
        acck = jnp.full((16,), -1, jnp.int32)
        accm = jnp.full((16,), 0, jnp.int32)
        for src in range(NS):
            k = kt_v[pl.ds(src * UPT + h * 16, 16)]
            m = mt_v[pl.ds(src * UPT + h * 16, 16)]
            better = k > acck
            acck = jnp.where(better, k, acck)
            accm = jnp.where(better, m, accm)
        sel_v[pl.ds(h * 16, 16)] = accm >> 8
        rnk_v[pl.ds(h * 16, 16)] = (accm >> 4) & 15
        augs.append(accm & 15)

    small_cps = [
        pltpu.async_copy(sel_v, out_idx_hbm.at[pl.ds(ubase, UPT)], sem),
        pltpu.async_copy(rnk_v, out_rank_hbm.at[pl.ds(ubase, UPT)], sem),
    ]

    exv = [ex_v[pl.ds(0, 16)], ex_v[pl.ds(16, 16)]]
    ebv = [e & -128 for e in exv]
    eov = [e & 127 for e in exv]
    blk_cps = []
    for i in range(UPT):
        aug_i = augs[i // 16][i % 16]
        eb_i = pl.multiple_of(ebv[i // 16][i % 16], 128)
        blk_cps.append(pltpu.async_copy(
            tea_hbm.at[aug_i + 1, :, pl.ds(eb_i, 128)], win_v.at[i], sem))
    for cp in blk_cps:
        cp.wait()

    zeros = jnp.zeros((16,), jnp.int32)

    def gather_body(c, _):
        for h in range(UPT // 16):
            vals = plsc.load_gather(win_v, [lane + h * 16, zeros + c, eov[h]])
            row_v[pl.ds(c * UPT + h * 16, 16)] = vals
        return 0

    lax.fori_loop(0, C, gather_body, 0)

    out_cps = []
    for c in range(C):
        out_cps.append(pltpu.async_copy(
            row_v.at[pl.ds(c * UPT, UPT)],
            out_tea_hbm.at[pl.ds(c * U + ubase, UPT)], sem))
    for cp in small_cps:
        cp.wait()
    for cp in out_cps:
        cp.wait()


def kernel(stu_logits, teacher_logits, augment_rank, nn_mask, example_indices,
           augmented_indices, nn_ranks):
    del stu_logits, augment_rank
    tea_t = jnp.transpose(teacher_logits, (1, 2, 0))
    sel_idx, sel_rank, sel_tea = _fused(
        nn_mask, nn_ranks, augmented_indices, example_indices, tea_t)
    return sel_idx, sel_rank, sel_tea.reshape(C, U).T


# speedup vs baseline: 19.9174x; 1.0040x over previous
import functools

import numpy as np
import jax
import jax.numpy as jnp
from jax import lax
from jax.experimental import pallas as pl
from jax.experimental.pallas import tpu as pltpu
from jax.experimental.pallas import tpu_sc as plsc

M = 16384
U = 1024
NC = 2
NS = 16
HALF = U // NC
EPT = M // NS
UPT = HALF // NS
C = 16

_KS0 = np.uint32(0)
_KS1 = np.uint32(42)
_KS2 = np.uint32(0x1BD11BDA ^ 42)
_ROT0 = (13, 15, 26, 6)
_ROT1 = (17, 29, 16, 24)


def _threefry_bits(flat):
    x0 = jnp.zeros_like(flat)
    x1 = flat + _KS1
    ks = (_KS0, _KS1, _KS2)
    for g in range(5):
        for r in (_ROT0 if g % 2 == 0 else _ROT1):
            x0 = x0 + x1
            x1 = (x1 << np.uint32(r)) | (x1 >> np.uint32(32 - r))
            x1 = x1 ^ x0
        x0 = x0 + ks[(g + 1) % 3]
        x1 = x1 + ks[(g + 2) % 3] + np.uint32(g + 1)
    return x0 ^ x1


_MESH = plsc.VectorSubcoreMesh(core_axis_name="c", subcore_axis_name="s")
_PARAMS = pltpu.CompilerParams(
    needs_layout_passes=False,
    disable_bounds_checks=True,
    disable_semaphore_checks=True,
)


@functools.partial(
    pl.kernel,
    mesh=_MESH,
    compiler_params=_PARAMS,
    out_type=(
        jax.ShapeDtypeStruct((U,), jnp.int32),
        jax.ShapeDtypeStruct((U,), jnp.int32),
        jax.ShapeDtypeStruct((U * C,), jnp.float32),
    ),
    scratch_types=[
        pltpu.VMEM((EPT,), jnp.int32),
        pltpu.VMEM((EPT,), jnp.int32),
        pltpu.VMEM((EPT,), jnp.int32),
        pltpu.VMEM((16 * HALF,), jnp.int32),
        pltpu.VMEM((16 * HALF,), jnp.int32),
        pltpu.VMEM((NS * UPT,), jnp.int32),
        pltpu.VMEM((NS * UPT,), jnp.int32),
        pltpu.VMEM((UPT,), jnp.int32),
        pltpu.VMEM((UPT,), jnp.int32),
        pltpu.VMEM((UPT,), jnp.int32),
        pltpu.VMEM((UPT, C, 128), jnp.float32),
        pltpu.VMEM((C * UPT,), jnp.float32),
        pltpu.VMEM_SHARED((NS, HALF), jnp.int32),
        pltpu.VMEM_SHARED((NS, HALF), jnp.int32),
        pltpu.SemaphoreType.DMA,
    ],
)
def _fused(nn_hbm, ranks_hbm, aug_hbm, ex_hbm, tea_hbm,
           out_idx_hbm, out_rank_hbm, out_tea_hbm,
           mask_v, rank_v, augc_v, tk_v, tm_v, kt_v, mt_v,
           sel_v, rnk_v, ex_v, win_v, row_v, shk_s, shm_s, sem):
    cid = lax.axis_index("c")
    sid = lax.axis_index("s")
    base = sid * EPT
    lubase = sid * UPT
    ubase = cid * HALF + lubase
    in_cps = [
        pltpu.async_copy(nn_hbm.at[pl.ds(base, EPT)], mask_v, sem),
        pltpu.async_copy(ranks_hbm.at[pl.ds(base, EPT)], rank_v, sem),
        pltpu.async_copy(aug_hbm.at[pl.ds(base, EPT)], augc_v, sem),
        pltpu.async_copy(ex_hbm.at[pl.ds(ubase, UPT)], ex_v, sem),
    ]

    lane = lax.iota(jnp.int32, 16)
    neg1 = jnp.full((16,), -1, jnp.int32)

    def init_body(i, _):
        for r in range(8):
            tk_v[pl.ds(i * 128 + r * 16, 16)] = neg1
        return 0

    lax.fori_loop(0, 16 * HALF // 128, init_body, 0)
    for cp in in_cps:
        cp.wait()

    def scatter_body(j, _):
        u16 = mask_v[pl.ds(j * 16, 16)]
        mglob = base + j * 16 + lane
        flat = (u16 * M + mglob).astype(jnp.uint32)
        k16 = (_threefry_bits(flat) >> np.uint32(9)).astype(jnp.int32)
        meta = (mglob << 8) | (rank_v[pl.ds(j * 16, 16)] << 4) \
            | augc_v[pl.ds(j * 16, 16)]
        inhalf = (u16 >> 9) == cid
        gidx = lane * HALF + (u16 & (HALF - 1))
        cur_k = plsc.load_gather(tk_v, [gidx])
        better = inhalf & (k16 > cur_k)
        plsc.store_scatter(tk_v, [gidx], k16, mask=better)
        plsc.store_scatter(tm_v, [gidx], meta, mask=better)
        return 0

    lax.fori_loop(0, EPT // 16, scatter_body, 0)

    for s in (8, 4, 2, 1):
        def fold_body(c, _, s=s):
            for l in range(s):
                a = l * HALF + c * 16
                b = (l + s) * HALF + c * 16
                ka = tk_v[pl.ds(a, 16)]
                kb = tk_v[pl.ds(b, 16)]
                ma = tm_v[pl.ds(a, 16)]
                mb = tm_v[pl.ds(b, 16)]
                better = (kb > ka) | ((kb == ka) & (mb < ma))
                tk_v[pl.ds(a, 16)] = jnp.where(better, kb, ka)
                tm_v[pl.ds(a, 16)] = jnp.where(better, mb, ma)
            return 0

        lax.fori_loop(0, HALF // 16, fold_body, 0)

    pltpu.sync_copy(tk_v.at[pl.ds(0, HALF)], shk_s.at[sid])
    pltpu.sync_copy(tm_v.at[pl.ds(0, HALF)], shm_s.at[sid])
    plsc.subcore_barrier()

    tab_cps = []
    for src in range(NS):
        tab_cps.append(pltpu.async_copy(
            shk_s.at[src, pl.ds(lubase, UPT)],
            kt_v.at[pl.ds(src * UPT, UPT)], sem))
        tab_cps.append(pltpu.async_copy(
            shm_s.at[src, pl.ds(lubase, UPT)],
            mt_v.at[pl.ds(src * UPT, UPT)], sem))
    for cp in tab_cps:
        cp.wait()

    zero16 = jnp.zeros((16,), jnp.int32)

    def merge_body(src, carry):
        acck0, accm0, acck1, accm1 = carry
        k0 = kt_v[pl.ds(src * UPT, 16)]
        m0 = mt_v[pl.ds(src * UPT, 16)]
        k1 = kt_v[pl.ds(src * UPT + 16, 16)]
        m1 = mt_v[pl.ds(src * UPT + 16, 16)]
        b0 = k0 > acck0
        b1 = k1 > acck1
        return (jnp.where(b0, k0, acck0), jnp.where(b0, m0, accm0),
                jnp.where(b1, k1, acck1), jnp.where(b1, m1, accm1))

    _, accm0, _, accm1 = lax.fori_loop(
        0, NS, merge_body, (neg1, zero16, neg1, zero16))
    augs = []
    for h, accm in enumerate((accm0, accm1)):
        sel_v[pl.ds(h * 16, 16)] = accm >> 8
        rnk_v[pl.ds(h * 16, 16)] = (accm >> 4) & 15
        augs.append(accm & 15)

    small_cps = [
        pltpu.async_copy(sel_v, out_idx_hbm.at[pl.ds(ubase, UPT)], sem),
        pltpu.async_copy(rnk_v, out_rank_hbm.at[pl.ds(ubase, UPT)], sem),
    ]

    exv = [ex_v[pl.ds(0, 16)], ex_v[pl.ds(16, 16)]]
    ebv = [e & -128 for e in exv]
    eov = [e & 127 for e in exv]
    blk_cps = []
    for i in range(UPT):
        aug_i = augs[i // 16][i % 16]
        eb_i = pl.multiple_of(ebv[i // 16][i % 16], 128)
        blk_cps.append(pltpu.async_copy(
            tea_hbm.at[aug_i + 1, :, pl.ds(eb_i, 128)], win_v.at[i], sem))
    for cp in blk_cps:
        cp.wait()

    zeros = jnp.zeros((16,), jnp.int32)

    def gather_body(c, _):
        for h in range(UPT // 16):
            vals = plsc.load_gather(win_v, [lane + h * 16, zeros + c, eov[h]])
            row_v[pl.ds(c * UPT + h * 16, 16)] = vals
        return 0

    lax.fori_loop(0, C, gather_body, 0)

    out_cps = []
    for c in range(C):
        out_cps.append(pltpu.async_copy(
            row_v.at[pl.ds(c * UPT, UPT)],
            out_tea_hbm.at[pl.ds(c * U + ubase, UPT)], sem))
    for cp in small_cps:
        cp.wait()
    for cp in out_cps:
        cp.wait()


def kernel(stu_logits, teacher_logits, augment_rank, nn_mask, example_indices,
           augmented_indices, nn_ranks):
    del stu_logits, augment_rank
    tea_t = jnp.transpose(teacher_logits, (1, 2, 0))
    sel_idx, sel_rank, sel_tea = _fused(
        nn_mask, nn_ranks, augmented_indices, example_indices, tea_t)
    return sel_idx, sel_rank, sel_tea.reshape(C, U).T


# speedup vs baseline: 20.5586x; 1.0322x over previous
import functools

import numpy as np
import jax
import jax.numpy as jnp
from jax import lax
from jax.experimental import pallas as pl
from jax.experimental.pallas import tpu as pltpu
from jax.experimental.pallas import tpu_sc as plsc

M = 16384
U = 1024
NC = 2
NS = 16
HALF = U // NC
EPT = M // NS
UPT = HALF // NS
C = 16

_KS0 = np.uint32(0)
_KS1 = np.uint32(42)
_KS2 = np.uint32(0x1BD11BDA ^ 42)
_ROT0 = (13, 15, 26, 6)
_ROT1 = (17, 29, 16, 24)


def _threefry_bits(flat):
    x0 = jnp.zeros_like(flat)
    x1 = flat + _KS1
    ks = (_KS0, _KS1, _KS2)
    for g in range(5):
        for r in (_ROT0 if g % 2 == 0 else _ROT1):
            x0 = x0 + x1
            x1 = (x1 << np.uint32(r)) | (x1 >> np.uint32(32 - r))
            x1 = x1 ^ x0
        x0 = x0 + ks[(g + 1) % 3]
        x1 = x1 + ks[(g + 2) % 3] + np.uint32(g + 1)
    return x0 ^ x1


_MESH = plsc.VectorSubcoreMesh(core_axis_name="c", subcore_axis_name="s")
_PARAMS = pltpu.CompilerParams(
    needs_layout_passes=False,
    disable_bounds_checks=True,
    disable_semaphore_checks=True,
)


@functools.partial(
    pl.kernel,
    mesh=_MESH,
    compiler_params=_PARAMS,
    out_type=(
        jax.ShapeDtypeStruct((U,), jnp.int32),
        jax.ShapeDtypeStruct((U,), jnp.int32),
        jax.ShapeDtypeStruct((U * C,), jnp.float32),
    ),
    scratch_types=[
        pltpu.VMEM((EPT,), jnp.int32),
        pltpu.VMEM((EPT,), jnp.int32),
        pltpu.VMEM((EPT,), jnp.int32),
        pltpu.VMEM((16 * HALF,), jnp.int32),
        pltpu.VMEM((16 * HALF,), jnp.int32),
        pltpu.VMEM((NS * UPT,), jnp.int32),
        pltpu.VMEM((NS * UPT,), jnp.int32),
        pltpu.VMEM((UPT,), jnp.int32),
        pltpu.VMEM((UPT,), jnp.int32),
        pltpu.VMEM((UPT,), jnp.int32),
        pltpu.VMEM((UPT, C, 128), jnp.float32),
        pltpu.VMEM((C * UPT,), jnp.float32),
        pltpu.VMEM_SHARED((NS, HALF), jnp.int32),
        pltpu.VMEM_SHARED((NS, HALF), jnp.int32),
        pltpu.SemaphoreType.DMA,
    ],
)
def _fused(nn_hbm, ranks_hbm, aug_hbm, ex_hbm, tea_hbm,
           out_idx_hbm, out_rank_hbm, out_tea_hbm,
           mask_v, rank_v, augc_v, tk_v, tm_v, kt_v, mt_v,
           sel_v, rnk_v, ex_v, win_v, row_v, shk_s, shm_s, sem):
    cid = lax.axis_index("c")
    sid = lax.axis_index("s")
    base = sid * EPT
    lubase = sid * UPT
    ubase = cid * HALF + lubase
    in_cps = [
        pltpu.async_copy(nn_hbm.at[pl.ds(base, EPT)], mask_v, sem),
        pltpu.async_copy(ranks_hbm.at[pl.ds(base, EPT)], rank_v, sem),
        pltpu.async_copy(aug_hbm.at[pl.ds(base, EPT)], augc_v, sem),
        pltpu.async_copy(ex_hbm.at[pl.ds(ubase, UPT)], ex_v, sem),
    ]

    lane = lax.iota(jnp.int32, 16)
    neg1 = jnp.full((16,), -1, jnp.int32)

    def init_body(i, _):
        for r in range(8):
            tk_v[pl.ds(i * 128 + r * 16, 16)] = neg1
        return 0

    lax.fori_loop(0, 16 * HALF // 128, init_body, 0)
    for cp in in_cps:
        cp.wait()

    def scatter_body(j, _):
        us, ks, metas, gidxs = [], [], [], []
        for t in range(2):
            u16 = mask_v[pl.ds(j * 32 + t * 16, 16)]
            mglob = base + j * 32 + t * 16 + lane
            flat = (u16 * M + mglob).astype(jnp.uint32)
            k16 = (_threefry_bits(flat) >> np.uint32(9)).astype(jnp.int32)
            meta = (mglob << 8) | (rank_v[pl.ds(j * 32 + t * 16, 16)] << 4) \
                | augc_v[pl.ds(j * 32 + t * 16, 16)]
            us.append(u16)
            ks.append(k16)
            metas.append(meta)
            gidxs.append(lane * HALF + (u16 & (HALF - 1)))
        for t in range(2):
            inhalf = (us[t] >> 9) == cid
            cur_k = plsc.load_gather(tk_v, [gidxs[t]])
            better = inhalf & (ks[t] > cur_k)
            plsc.store_scatter(tk_v, [gidxs[t]], ks[t], mask=better)
            plsc.store_scatter(tm_v, [gidxs[t]], metas[t], mask=better)
        return 0

    lax.fori_loop(0, EPT // 32, scatter_body, 0)

    for s in (8, 4, 2, 1):
        def fold_body(c, _, s=s):
            for l in range(s):
                a = l * HALF + c * 16
                b = (l + s) * HALF + c * 16
                ka = tk_v[pl.ds(a, 16)]
                kb = tk_v[pl.ds(b, 16)]
                ma = tm_v[pl.ds(a, 16)]
                mb = tm_v[pl.ds(b, 16)]
                better = (kb > ka) | ((kb == ka) & (mb < ma))
                tk_v[pl.ds(a, 16)] = jnp.where(better, kb, ka)
                tm_v[pl.ds(a, 16)] = jnp.where(better, mb, ma)
            return 0

        lax.fori_loop(0, HALF // 16, fold_body, 0)

    pltpu.sync_copy(tk_v.at[pl.ds(0, HALF)], shk_s.at[sid])
    pltpu.sync_copy(tm_v.at[pl.ds(0, HALF)], shm_s.at[sid])
    plsc.subcore_barrier()

    tab_cps = []
    for src in range(NS):
        tab_cps.append(pltpu.async_copy(
            shk_s.at[src, pl.ds(lubase, UPT)],
            kt_v.at[pl.ds(src * UPT, UPT)], sem))
        tab_cps.append(pltpu.async_copy(
            shm_s.at[src, pl.ds(lubase, UPT)],
            mt_v.at[pl.ds(src * UPT, UPT)], sem))
    for cp in tab_cps:
        cp.wait()

    zero16 = jnp.zeros((16,), jnp.int32)

    def merge_body(src, carry):
        acck0, accm0, acck1, accm1 = carry
        k0 = kt_v[pl.ds(src * UPT, 16)]
        m0 = mt_v[pl.ds(src * UPT, 16)]
        k1 = kt_v[pl.ds(src * UPT + 16, 16)]
        m1 = mt_v[pl.ds(src * UPT + 16, 16)]
        b0 = k0 > acck0
        b1 = k1 > acck1
        return (jnp.where(b0, k0, acck0), jnp.where(b0, m0, accm0),
                jnp.where(b1, k1, acck1), jnp.where(b1, m1, accm1))

    _, accm0, _, accm1 = lax.fori_loop(
        0, NS, merge_body, (neg1, zero16, neg1, zero16))
    augs = []
    for h, accm in enumerate((accm0, accm1)):
        sel_v[pl.ds(h * 16, 16)] = accm >> 8
        rnk_v[pl.ds(h * 16, 16)] = (accm >> 4) & 15
        augs.append(accm & 15)

    small_cps = [
        pltpu.async_copy(sel_v, out_idx_hbm.at[pl.ds(ubase, UPT)], sem),
        pltpu.async_copy(rnk_v, out_rank_hbm.at[pl.ds(ubase, UPT)], sem),
    ]

    exv = [ex_v[pl.ds(0, 16)], ex_v[pl.ds(16, 16)]]
    ebv = [e & -128 for e in exv]
    eov = [e & 127 for e in exv]
    blk_cps = []
    for i in range(UPT):
        aug_i = augs[i // 16][i % 16]
        eb_i = pl.multiple_of(ebv[i // 16][i % 16], 128)
        blk_cps.append(pltpu.async_copy(
            tea_hbm.at[aug_i + 1, :, pl.ds(eb_i, 128)], win_v.at[i], sem))
    for cp in blk_cps:
        cp.wait()

    zeros = jnp.zeros((16,), jnp.int32)

    def gather_body(c, _):
        for h in range(UPT // 16):
            vals = plsc.load_gather(win_v, [lane + h * 16, zeros + c, eov[h]])
            row_v[pl.ds(c * UPT + h * 16, 16)] = vals
        return 0

    lax.fori_loop(0, C, gather_body, 0)

    out_cps = []
    for c in range(C):
        out_cps.append(pltpu.async_copy(
            row_v.at[pl.ds(c * UPT, UPT)],
            out_tea_hbm.at[pl.ds(c * U + ubase, UPT)], sem))
    for cp in small_cps:
        cp.wait()
    for cp in out_cps:
        cp.wait()


def kernel(stu_logits, teacher_logits, augment_rank, nn_mask, example_indices,
           augmented_indices, nn_ranks):
    del stu_logits, augment_rank
    tea_t = jnp.transpose(teacher_logits, (1, 2, 0))
    sel_idx, sel_rank, sel_tea = _fused(
        nn_mask, nn_ranks, augmented_indices, example_indices, tea_t)
    return sel_idx, sel_rank, sel_tea.reshape(C, U).T


# speedup vs baseline: 20.6044x; 1.0022x over previous
import functools

import numpy as np
import jax
import jax.numpy as jnp
from jax import lax
from jax.experimental import pallas as pl
from jax.experimental.pallas import tpu as pltpu
from jax.experimental.pallas import tpu_sc as plsc

M = 16384
U = 1024
NC = 2
NS = 16
HALF = U // NC
EPT = M // NS
UPT = HALF // NS
C = 16

_KS0 = np.uint32(0)
_KS1 = np.uint32(42)
_KS2 = np.uint32(0x1BD11BDA ^ 42)
_ROT0 = (13, 15, 26, 6)
_ROT1 = (17, 29, 16, 24)


def _threefry_bits(flat):
    x0 = jnp.zeros_like(flat)
    x1 = flat + _KS1
    ks = (_KS0, _KS1, _KS2)
    for g in range(5):
        for r in (_ROT0 if g % 2 == 0 else _ROT1):
            x0 = x0 + x1
            x1 = (x1 << np.uint32(r)) | (x1 >> np.uint32(32 - r))
            x1 = x1 ^ x0
        x0 = x0 + ks[(g + 1) % 3]
        x1 = x1 + ks[(g + 2) % 3] + np.uint32(g + 1)
    return x0 ^ x1


_MESH = plsc.VectorSubcoreMesh(core_axis_name="c", subcore_axis_name="s")
_PARAMS = pltpu.CompilerParams(
    needs_layout_passes=False,
    disable_bounds_checks=True,
    disable_semaphore_checks=True,
)


@functools.partial(
    pl.kernel,
    mesh=_MESH,
    compiler_params=_PARAMS,
    out_type=(
        jax.ShapeDtypeStruct((U,), jnp.int32),
        jax.ShapeDtypeStruct((U,), jnp.int32),
        jax.ShapeDtypeStruct((U * C,), jnp.float32),
    ),
    scratch_types=[
        pltpu.VMEM((EPT,), jnp.int32),
        pltpu.VMEM((EPT,), jnp.int32),
        pltpu.VMEM((EPT,), jnp.int32),
        pltpu.VMEM((16 * HALF,), jnp.int32),
        pltpu.VMEM((16 * HALF,), jnp.int32),
        pltpu.VMEM((NS * UPT,), jnp.int32),
        pltpu.VMEM((NS * UPT,), jnp.int32),
        pltpu.VMEM((UPT,), jnp.int32),
        pltpu.VMEM((UPT,), jnp.int32),
        pltpu.VMEM((UPT,), jnp.int32),
        pltpu.VMEM((UPT, C, 128), jnp.float32),
        pltpu.VMEM((C * UPT,), jnp.float32),
        pltpu.VMEM_SHARED((NS, HALF), jnp.int32),
        pltpu.VMEM_SHARED((NS, HALF), jnp.int32),
        pltpu.SemaphoreType.DMA,
    ],
)
def _fused(nn_hbm, ranks_hbm, aug_hbm, ex_hbm, tea_hbm,
           out_idx_hbm, out_rank_hbm, out_tea_hbm,
           mask_v, rank_v, augc_v, tk_v, tm_v, kt_v, mt_v,
           sel_v, rnk_v, ex_v, win_v, row_v, shk_s, shm_s, sem):
    cid = lax.axis_index("c")
    sid = lax.axis_index("s")
    base = sid * EPT
    lubase = sid * UPT
    ubase = cid * HALF + lubase
    in_cps = [
        pltpu.async_copy(nn_hbm.at[pl.ds(base, EPT)], mask_v, sem),
        pltpu.async_copy(ranks_hbm.at[pl.ds(base, EPT)], rank_v, sem),
        pltpu.async_copy(aug_hbm.at[pl.ds(base, EPT)], augc_v, sem),
        pltpu.async_copy(ex_hbm.at[pl.ds(ubase, UPT)], ex_v, sem),
    ]

    lane = lax.iota(jnp.int32, 16)
    neg1 = jnp.full((16,), -1, jnp.int32)

    def init_body(i, _):
        for r in range(8):
            tk_v[pl.ds(i * 128 + r * 16, 16)] = neg1
        return 0

    lax.fori_loop(0, 16 * HALF // 128, init_body, 0)
    for cp in in_cps:
        cp.wait()

    def scatter_body(j, _):
        us, ks, metas, gidxs = [], [], [], []
        for t in range(4):
            u16 = mask_v[pl.ds(j * 64 + t * 16, 16)]
            mglob = base + j * 64 + t * 16 + lane
            flat = (u16 * M + mglob).astype(jnp.uint32)
            k16 = (_threefry_bits(flat) >> np.uint32(9)).astype(jnp.int32)
            meta = (mglob << 8) | (rank_v[pl.ds(j * 64 + t * 16, 16)] << 4) \
                | augc_v[pl.ds(j * 64 + t * 16, 16)]
            us.append(u16)
            ks.append(k16)
            metas.append(meta)
            gidxs.append(lane * HALF + (u16 & (HALF - 1)))
        for t in range(4):
            inhalf = (us[t] >> 9) == cid
            cur_k = plsc.load_gather(tk_v, [gidxs[t]])
            better = inhalf & (ks[t] > cur_k)
            plsc.store_scatter(tk_v, [gidxs[t]], ks[t], mask=better)
            plsc.store_scatter(tm_v, [gidxs[t]], metas[t], mask=better)
        return 0

    lax.fori_loop(0, EPT // 64, scatter_body, 0)

    for s in (8, 4, 2, 1):
        def fold_body(c, _, s=s):
            for l in range(s):
                a = l * HALF + c * 16
                b = (l + s) * HALF + c * 16
                ka = tk_v[pl.ds(a, 16)]
                kb = tk_v[pl.ds(b, 16)]
                ma = tm_v[pl.ds(a, 16)]
                mb = tm_v[pl.ds(b, 16)]
                better = (kb > ka) | ((kb == ka) & (mb < ma))
                tk_v[pl.ds(a, 16)] = jnp.where(better, kb, ka)
                tm_v[pl.ds(a, 16)] = jnp.where(better, mb, ma)
            return 0

        lax.fori_loop(0, HALF // 16, fold_body, 0)

    pltpu.sync_copy(tk_v.at[pl.ds(0, HALF)], shk_s.at[sid])
    pltpu.sync_copy(tm_v.at[pl.ds(0, HALF)], shm_s.at[sid])
    plsc.subcore_barrier()

    tab_cps = []
    for src in range(NS):
        tab_cps.append(pltpu.async_copy(
            shk_s.at[src, pl.ds(lubase, UPT)],
            kt_v.at[pl.ds(src * UPT, UPT)], sem))
        tab_cps.append(pltpu.async_copy(
            shm_s.at[src, pl.ds(lubase, UPT)],
            mt_v.at[pl.ds(src * UPT, UPT)], sem))
    for cp in tab_cps:
        cp.wait()

    zero16 = jnp.zeros((16,), jnp.int32)

    def merge_body(src, carry):
        acck0, accm0, acck1, accm1 = carry
        k0 = kt_v[pl.ds(src * UPT, 16)]
        m0 = mt_v[pl.ds(src * UPT, 16)]
        k1 = kt_v[pl.ds(src * UPT + 16, 16)]
        m1 = mt_v[pl.ds(src * UPT + 16, 16)]
        b0 = k0 > acck0
        b1 = k1 > acck1
        return (jnp.where(b0, k0, acck0), jnp.where(b0, m0, accm0),
                jnp.where(b1, k1, acck1), jnp.where(b1, m1, accm1))

    _, accm0, _, accm1 = lax.fori_loop(
        0, NS, merge_body, (neg1, zero16, neg1, zero16))
    augs = []
    for h, accm in enumerate((accm0, accm1)):
        sel_v[pl.ds(h * 16, 16)] = accm >> 8
        rnk_v[pl.ds(h * 16, 16)] = (accm >> 4) & 15
        augs.append(accm & 15)

    small_cps = [
        pltpu.async_copy(sel_v, out_idx_hbm.at[pl.ds(ubase, UPT)], sem),
        pltpu.async_copy(rnk_v, out_rank_hbm.at[pl.ds(ubase, UPT)], sem),
    ]

    exv = [ex_v[pl.ds(0, 16)], ex_v[pl.ds(16, 16)]]
    ebv = [e & -128 for e in exv]
    eov = [e & 127 for e in exv]
    blk_cps = []
    for i in range(UPT):
        aug_i = augs[i // 16][i % 16]
        eb_i = pl.multiple_of(ebv[i // 16][i % 16], 128)
        blk_cps.append(pltpu.async_copy(
            tea_hbm.at[aug_i + 1, :, pl.ds(eb_i, 128)], win_v.at[i], sem))
    for cp in blk_cps:
        cp.wait()

    zeros = jnp.zeros((16,), jnp.int32)

    def gather_body(c, _):
        for h in range(UPT // 16):
            vals = plsc.load_gather(win_v, [lane + h * 16, zeros + c, eov[h]])
            row_v[pl.ds(c * UPT + h * 16, 16)] = vals
        return 0

    lax.fori_loop(0, C, gather_body, 0)

    out_cps = []
    for c in range(C):
        out_cps.append(pltpu.async_copy(
            row_v.at[pl.ds(c * UPT, UPT)],
            out_tea_hbm.at[pl.ds(c * U + ubase, UPT)], sem))
    for cp in small_cps:
        cp.wait()
    for cp in out_cps:
        cp.wait()


def kernel(stu_logits, teacher_logits, augment_rank, nn_mask, example_indices,
           augmented_indices, nn_ranks):
    del stu_logits, augment_rank
    tea_t = jnp.transpose(teacher_logits, (1, 2, 0))
    sel_idx, sel_rank, sel_tea = _fused(
        nn_mask, nn_ranks, augmented_indices, example_indices, tea_t)
    return sel_idx, sel_rank, sel_tea.reshape(C, U).T


# speedup vs baseline: 20.6141x; 1.0005x over previous
import functools

import numpy as np
import jax
import jax.numpy as jnp
from jax import lax
from jax.experimental import pallas as pl
from jax.experimental.pallas import tpu as pltpu
from jax.experimental.pallas import tpu_sc as plsc

M = 16384
U = 1024
NC = 2
NS = 16
HALF = U // NC
EPT = M // NS
UPT = HALF // NS
C = 16

_KS0 = np.uint32(0)
_KS1 = np.uint32(42)
_KS2 = np.uint32(0x1BD11BDA ^ 42)
_ROT0 = (13, 15, 26, 6)
_ROT1 = (17, 29, 16, 24)


def _threefry_bits(flat):
    x0 = jnp.zeros_like(flat)
    x1 = flat + _KS1
    ks = (_KS0, _KS1, _KS2)
    for g in range(5):
        for r in (_ROT0 if g % 2 == 0 else _ROT1):
            x0 = x0 + x1
            x1 = (x1 << np.uint32(r)) | (x1 >> np.uint32(32 - r))
            x1 = x1 ^ x0
        x0 = x0 + ks[(g + 1) % 3]
        x1 = x1 + ks[(g + 2) % 3] + np.uint32(g + 1)
    return x0 ^ x1


_MESH = plsc.VectorSubcoreMesh(core_axis_name="c", subcore_axis_name="s")
_PARAMS = pltpu.CompilerParams(
    needs_layout_passes=False,
    disable_bounds_checks=True,
    disable_semaphore_checks=True,
)


@functools.partial(
    pl.kernel,
    mesh=_MESH,
    compiler_params=_PARAMS,
    out_type=(
        jax.ShapeDtypeStruct((U,), jnp.int32),
        jax.ShapeDtypeStruct((U,), jnp.int32),
        jax.ShapeDtypeStruct((U * C,), jnp.float32),
    ),
    scratch_types=[
        pltpu.VMEM((EPT,), jnp.int32),
        pltpu.VMEM((EPT,), jnp.int32),
        pltpu.VMEM((EPT,), jnp.int32),
        pltpu.VMEM((16 * HALF,), jnp.int32),
        pltpu.VMEM((16 * HALF,), jnp.int32),
        pltpu.VMEM((NS * UPT,), jnp.int32),
        pltpu.VMEM((NS * UPT,), jnp.int32),
        pltpu.VMEM((UPT,), jnp.int32),
        pltpu.VMEM((UPT,), jnp.int32),
        pltpu.VMEM((UPT,), jnp.int32),
        pltpu.VMEM((UPT, C, 128), jnp.float32),
        pltpu.VMEM((C * UPT,), jnp.float32),
        pltpu.VMEM_SHARED((NS, HALF), jnp.int32),
        pltpu.VMEM_SHARED((NS, HALF), jnp.int32),
        pltpu.SemaphoreType.DMA,
    ],
)
def _fused(nn_hbm, ranks_hbm, aug_hbm, ex_hbm, tea_hbm,
           out_idx_hbm, out_rank_hbm, out_tea_hbm,
           mask_v, rank_v, augc_v, tk_v, tm_v, kt_v, mt_v,
           sel_v, rnk_v, ex_v, win_v, row_v, shk_s, shm_s, sem):
    cid = lax.axis_index("c")
    sid = lax.axis_index("s")
    base = sid * EPT
    lubase = sid * UPT
    ubase = cid * HALF + lubase
    in_cps = [
        pltpu.async_copy(nn_hbm.at[pl.ds(base, EPT)], mask_v, sem),
        pltpu.async_copy(ranks_hbm.at[pl.ds(base, EPT)], rank_v, sem),
        pltpu.async_copy(aug_hbm.at[pl.ds(base, EPT)], augc_v, sem),
        pltpu.async_copy(ex_hbm.at[pl.ds(ubase, UPT)], ex_v, sem),
    ]

    lane = lax.iota(jnp.int32, 16)
    neg1 = jnp.full((16,), -1, jnp.int32)

    def init_body(i, _):
        for r in range(8):
            tk_v[pl.ds(i * 128 + r * 16, 16)] = neg1
        return 0

    lax.fori_loop(0, 16 * HALF // 128, init_body, 0)
    for cp in in_cps:
        cp.wait()

    def scatter_body(j, _):
        us, ks, metas, gidxs = [], [], [], []
        for t in range(2):
            u16 = mask_v[pl.ds(j * 32 + t * 16, 16)]
            mglob = base + j * 32 + t * 16 + lane
            flat = (u16 * M + mglob).astype(jnp.uint32)
            k16 = (_threefry_bits(flat) >> np.uint32(9)).astype(jnp.int32)
            meta = (mglob << 8) | (rank_v[pl.ds(j * 32 + t * 16, 16)] << 4) \
                | augc_v[pl.ds(j * 32 + t * 16, 16)]
            us.append(u16)
            ks.append(k16)
            metas.append(meta)
            gidxs.append(lane * HALF + (u16 & (HALF - 1)))
        for t in range(2):
            inhalf = (us[t] >> 9) == cid
            cur_k = plsc.load_gather(tk_v, [gidxs[t]])
            better = inhalf & (ks[t] > cur_k)
            plsc.store_scatter(tk_v, [gidxs[t]], ks[t], mask=better)
            plsc.store_scatter(tm_v, [gidxs[t]], metas[t], mask=better)
        return 0

    lax.fori_loop(0, EPT // 32, scatter_body, 0)

    for s in (8, 4, 2, 1):
        def fold_body(c, _, s=s):
            for l in range(s):
                a = l * HALF + c * 16
                b = (l + s) * HALF + c * 16
                ka = tk_v[pl.ds(a, 16)]
                kb = tk_v[pl.ds(b, 16)]
                ma = tm_v[pl.ds(a, 16)]
                mb = tm_v[pl.ds(b, 16)]
                better = (kb > ka) | ((kb == ka) & (mb < ma))
                tk_v[pl.ds(a, 16)] = jnp.where(better, kb, ka)
                tm_v[pl.ds(a, 16)] = jnp.where(better, mb, ma)
            return 0

        lax.fori_loop(0, HALF // 16, fold_body, 0)

    pltpu.sync_copy(tk_v.at[pl.ds(0, HALF)], shk_s.at[sid])
    pltpu.sync_copy(tm_v.at[pl.ds(0, HALF)], shm_s.at[sid])
    plsc.subcore_barrier()

    tab_cps = []
    for src in range(NS):
        tab_cps.append(pltpu.async_copy(
            shk_s.at[src, pl.ds(lubase, UPT)],
            kt_v.at[pl.ds(src * UPT, UPT)], sem))
        tab_cps.append(pltpu.async_copy(
            shm_s.at[src, pl.ds(lubase, UPT)],
            mt_v.at[pl.ds(src * UPT, UPT)], sem))
    for cp in tab_cps:
        cp.wait()

    zero16 = jnp.zeros((16,), jnp.int32)

    def merge_body(src, carry):
        acck0, accm0, acck1, accm1 = carry
        k0 = kt_v[pl.ds(src * UPT, 16)]
        m0 = mt_v[pl.ds(src * UPT, 16)]
        k1 = kt_v[pl.ds(src * UPT + 16, 16)]
        m1 = mt_v[pl.ds(src * UPT + 16, 16)]
        b0 = k0 > acck0
        b1 = k1 > acck1
        return (jnp.where(b0, k0, acck0), jnp.where(b0, m0, accm0),
                jnp.where(b1, k1, acck1), jnp.where(b1, m1, accm1))

    _, accm0, _, accm1 = lax.fori_loop(
        0, NS, merge_body, (neg1, zero16, neg1, zero16))
    augs = []
    for h, accm in enumerate((accm0, accm1)):
        sel_v[pl.ds(h * 16, 16)] = accm >> 8
        rnk_v[pl.ds(h * 16, 16)] = (accm >> 4) & 15
        augs.append(accm & 15)

    small_cps = [
        pltpu.async_copy(sel_v, out_idx_hbm.at[pl.ds(ubase, UPT)], sem),
        pltpu.async_copy(rnk_v, out_rank_hbm.at[pl.ds(ubase, UPT)], sem),
    ]

    exv = [ex_v[pl.ds(0, 16)], ex_v[pl.ds(16, 16)]]
    ebv = [e & -128 for e in exv]
    eov = [e & 127 for e in exv]
    blk_cps = []
    for i in range(UPT):
        aug_i = augs[i // 16][i % 16]
        eb_i = pl.multiple_of(ebv[i // 16][i % 16], 128)
        blk_cps.append(pltpu.async_copy(
            tea_hbm.at[aug_i + 1, :, pl.ds(eb_i, 128)], win_v.at[i], sem))
    for cp in blk_cps:
        cp.wait()

    zeros = jnp.zeros((16,), jnp.int32)

    def gather_body(c, _):
        for h in range(UPT // 16):
            vals = plsc.load_gather(win_v, [lane + h * 16, zeros + c, eov[h]])
            row_v[pl.ds(c * UPT + h * 16, 16)] = vals
        return 0

    lax.fori_loop(0, C, gather_body, 0)

    out_cps = []
    for c in range(C):
        out_cps.append(pltpu.async_copy(
            row_v.at[pl.ds(c * UPT, UPT)],
            out_tea_hbm.at[pl.ds(c * U + ubase, UPT)], sem))
    for cp in small_cps:
        cp.wait()
    for cp in out_cps:
        cp.wait()


def kernel(stu_logits, teacher_logits, augment_rank, nn_mask, example_indices,
           augmented_indices, nn_ranks):
    del stu_logits, augment_rank
    tea_t = jnp.transpose(teacher_logits, (1, 2, 0))
    sel_idx, sel_rank, sel_tea = _fused(
        nn_mask, nn_ranks, augmented_indices, example_indices, tea_t)
    return sel_idx, sel_rank, sel_tea.reshape(C, U).T
